# baseline (speedup 1.0000x reference)
import functools

import jax
import jax.numpy as jnp
from jax import lax
from jax.experimental import pallas as pl
from jax.experimental.pallas import tpu as pltpu
from jax.experimental.pallas import tpu_sc as plsc

B, C, H, W = 4, 3136, 56, 56
S = H * W
CK = 392
NB = C // CK
K = 64


def _reduce_body(x_ref, ma_ref, val_ref):
    i = pl.program_id(1)
    x = x_ref[0]
    cmax = jnp.max(x, axis=0, keepdims=True)
    csum = jnp.sum(x, axis=0, keepdims=True)
    val_ref[0, 0] = jnp.sum(x, axis=1, keepdims=True)

    @pl.when(i == 0)
    def _():
        ma_ref[0] = jnp.concatenate([cmax, csum], axis=0)

    @pl.when(i > 0)
    def _():
        prev = ma_ref[0]
        ma_ref[0] = jnp.concatenate(
            [jnp.maximum(prev[0:1], cmax), prev[1:2] + csum], axis=0)

    @pl.when(i == NB - 1)
    def _():
        cur = ma_ref[0]
        ma_ref[0] = jnp.concatenate([cur[0:1], cur[1:2] * (1.0 / C)], axis=0)


def _topk_body(val_ref, sel_ref):
    v = val_ref[0]
    lanes = lax.broadcasted_iota(jnp.int32, (1, C), 1)
    lanes_k = lax.broadcasted_iota(jnp.int32, (1, K), 1)

    def body(j, carry):
        v, selv = carry
        m = jnp.max(v)
        idx = jnp.min(jnp.where(v >= m, lanes, C))
        selv = jnp.where(lanes_k == j, idx, selv)
        v = jnp.where(lanes == idx, -jnp.inf, v)
        return v, selv

    _, selv = lax.fori_loop(0, K, body, (v, jnp.zeros((1, K), jnp.int32)))
    sel_ref[0] = selv


def _sc_gather_body(corr_hbm, ma_hbm, sel_hbm, out_hbm, idx_v, rows_v, row_v, sem):
    w = lax.axis_index("s") * 2 + lax.axis_index("c")

    @pl.when(w < 16)
    def _():
        b = w // 4
        j0 = (w % 4) * 16
        pltpu.sync_copy(sel_hbm.at[pl.ds(b * K + j0, 16)], idx_v)
        idx_v[...] = idx_v[...] + b * C
        pltpu.async_copy(corr_hbm.at[idx_v], rows_v, sem).wait()
        pltpu.sync_copy(rows_v, out_hbm.at[pl.ds(b * 66 + 2 + j0, 16)])

    @pl.when((w >= 16) & (w < 24))
    def _():
        w2 = w - 16
        b = w2 // 2
        r = w2 % 2
        pltpu.sync_copy(ma_hbm.at[pl.ds(b * 2 + r, 1)], row_v)
        pltpu.sync_copy(row_v, out_hbm.at[pl.ds(b * 66 + r, 1)])


@functools.cache
def _sc_gather():
    return pl.kernel(
        _sc_gather_body,
        out_type=jax.ShapeDtypeStruct((B * 66, S), jnp.float32),
        mesh=plsc.VectorSubcoreMesh(core_axis_name="c", subcore_axis_name="s"),
        scratch_types=[
            pltpu.VMEM((16,), jnp.int32),
            pltpu.VMEM((16, S), jnp.float32),
            pltpu.VMEM((1, S), jnp.float32),
            pltpu.SemaphoreType.DMA,
        ],
        compiler_params=pltpu.CompilerParams(use_tc_tiling_on_sc=False),
    )


@jax.jit
def kernel(corr, select_indices):
    corr3 = corr.reshape(B, C, S)
    ma, val = pl.pallas_call(
        _reduce_body,
        grid=(B, NB),
        in_specs=[pl.BlockSpec((1, CK, S), lambda b, i: (b, i, 0))],
        out_specs=[
            pl.BlockSpec((1, 2, S), lambda b, i: (b, 0, 0)),
            pl.BlockSpec((1, 1, CK, 1), lambda b, i: (b, i, 0, 0)),
        ],
        out_shape=[
            jax.ShapeDtypeStruct((B, 2, S), jnp.float32),
            jax.ShapeDtypeStruct((B, NB, CK, 1), jnp.float32),
        ],
    )(corr3)

    value = val.reshape(B, 1, C)
    sel = pl.pallas_call(
        _topk_body,
        grid=(B,),
        in_specs=[pl.BlockSpec((1, 1, C), lambda b: (b, 0, 0))],
        out_specs=pl.BlockSpec((1, 1, K), lambda b: (b, 0, 0)),
        out_shape=jax.ShapeDtypeStruct((B, 1, K), jnp.int32),
    )(value)

    sel = jnp.take(sel.reshape(B, K), select_indices, axis=1)

    out = _sc_gather()(corr3.reshape(B * C, S), ma.reshape(B * 2, S),
                       sel.reshape(B * K).astype(jnp.int32))
    return out.reshape(B, 66, H, W)


# speedup vs baseline: 2.8155x; 2.8155x over previous
import functools

import jax
import jax.numpy as jnp
from jax import lax
from jax.experimental import pallas as pl
from jax.experimental.pallas import tpu as pltpu
from jax.experimental.pallas import tpu_sc as plsc

B, C, H, W = 4, 3136, 56, 56
CK = 392
NB = C // CK
K = 64


def _reduce_body(x_ref, ma_ref, val_ref):
    i = pl.program_id(1)
    x = x_ref[0]
    cmax = jnp.max(x, axis=0)
    csum = jnp.sum(x, axis=0)
    val_ref[0, 0] = jnp.sum(x, axis=(1, 2))[None, :]

    @pl.when(i == 0)
    def _():
        ma_ref[0, 0] = cmax
        ma_ref[0, 1] = csum

    @pl.when(i > 0)
    def _():
        ma_ref[0, 0] = jnp.maximum(ma_ref[0, 0], cmax)
        ma_ref[0, 1] = ma_ref[0, 1] + csum

    @pl.when(i == NB - 1)
    def _():
        ma_ref[0, 1] = ma_ref[0, 1] * (1.0 / C)


def _topk_body(val_ref, sel_ref):
    v = val_ref[...]
    lanes = lax.broadcasted_iota(jnp.int32, (B, C), 1)
    lanes_k = lax.broadcasted_iota(jnp.int32, (B, K), 1)

    def body(j, carry):
        v, selv = carry
        m = jnp.max(v, axis=1, keepdims=True)
        idx = jnp.min(jnp.where(v >= m, lanes, C), axis=1,
                      keepdims=True)
        selv = jnp.where(lanes_k == j, idx, selv)
        v = jnp.where(lanes == idx, -jnp.inf, v)
        return v, selv

    _, selv = lax.fori_loop(0, K, body,
                            (v, jnp.zeros((B, K), jnp.int32)))
    sel_ref[...] = selv


def _sc_gather_body(corr_hbm, ma_hbm, gsel_hbm, out_hbm, gsel_v, rows_v,
                    ma_v, sem):
    w = lax.axis_index("s") * 2 + lax.axis_index("c")
    b = w // 8
    j0 = (w % 8) * 8
    pltpu.sync_copy(gsel_hbm, gsel_v)
    g = gsel_v[pl.ds((w // 2) * 16, 16)]
    lane = lax.broadcasted_iota(jnp.int32, (16,), 0)
    off = (w % 2) * 8
    copies = []
    for j in range(8):
        idx_j = jnp.max(jnp.where(lane == off + j, g, -1))
        copies.append(pltpu.async_copy(
            corr_hbm.at[pl.ds(idx_j, 1)], rows_v.at[pl.ds(j, 1)], sem))
    for cp in copies:
        cp.wait()
    pltpu.sync_copy(rows_v, out_hbm.at[pl.ds(b * 66 + 2 + j0, 8)])

    @pl.when(w < 8)
    def _():
        pltpu.sync_copy(ma_hbm.at[pl.ds(w, 1)], ma_v)
        pltpu.sync_copy(ma_v, out_hbm.at[pl.ds((w // 2) * 66 + w % 2, 1)])


@functools.cache
def _sc_gather():
    return pl.kernel(
        _sc_gather_body,
        out_type=jax.ShapeDtypeStruct((B * 66, H, W), jnp.float32),
        mesh=plsc.VectorSubcoreMesh(core_axis_name="c", subcore_axis_name="s"),
        scratch_types=[
            pltpu.VMEM((B * K,), jnp.int32),
            pltpu.VMEM((8, H, W), jnp.float32),
            pltpu.VMEM((1, H, W), jnp.float32),
            pltpu.SemaphoreType.DMA,
        ],
        compiler_params=pltpu.CompilerParams(needs_layout_passes=False),
    )


@jax.jit
def kernel(corr, select_indices):
    ma, val = pl.pallas_call(
        _reduce_body,
        grid=(B, NB),
        in_specs=[pl.BlockSpec((1, CK, H, W), lambda b, i: (b, i, 0, 0))],
        out_specs=[
            pl.BlockSpec((1, 2, H, W), lambda b, i: (b, 0, 0, 0)),
            pl.BlockSpec((1, 1, 1, CK), lambda b, i: (b, i, 0, 0)),
        ],
        out_shape=[
            jax.ShapeDtypeStruct((B, 2, H, W), jnp.float32),
            jax.ShapeDtypeStruct((B, NB, 1, CK), jnp.float32),
        ],
    )(corr)

    sel = pl.pallas_call(
        _topk_body,
        grid=(1,),
        in_specs=[pl.BlockSpec((B, C), lambda i: (0, 0))],
        out_specs=pl.BlockSpec((B, K), lambda i: (0, 0)),
        out_shape=jax.ShapeDtypeStruct((B, K), jnp.int32),
    )(val.reshape(B, C))

    sel = jnp.take(sel, select_indices, axis=1)
    gsel = (sel + jnp.arange(B, dtype=jnp.int32)[:, None] * C).reshape(B * K)

    out = _sc_gather()(corr.reshape(B * C, H, W), ma.reshape(B * 2, H, W),
                       gsel.astype(jnp.int32))
    return out.reshape(B, 66, H, W)


# speedup vs baseline: 5.2662x; 1.8704x over previous
import functools

import jax
import jax.numpy as jnp
from jax import lax
from jax.experimental import pallas as pl
from jax.experimental.pallas import tpu as pltpu
from jax.experimental.pallas import tpu_sc as plsc

B, C, H, W = 4, 3136, 56, 56
HB = 8
NH = H // HB
K = 64
NPOS = B * H * W
NTIL = NPOS // 8
NWORK = 32
TPW = NTIL // NWORK


def _reduce_body(x_ref, ma_ref, val_ref):
    i = pl.program_id(1)
    x = x_ref[0]
    mx = jnp.max(x, axis=2)
    sm = jnp.sum(x, axis=2) * (1.0 / C)
    ma_ref[0] = jnp.stack([mx, sm], axis=-1)
    pv = jnp.sum(x, axis=(0, 1))[None, :]

    @pl.when(i == 0)
    def _():
        val_ref[0] = pv

    @pl.when(i > 0)
    def _():
        val_ref[0] = val_ref[0] + pv


def _topk_body(val_ref, sel_ref):
    v = val_ref[:, 0, :]
    lanes = lax.broadcasted_iota(jnp.int32, (B, C), 1)
    lanes_k = lax.broadcasted_iota(jnp.int32, (B, K), 1)

    def body(j, carry):
        v, selv = carry
        m = jnp.max(v, axis=1, keepdims=True)
        idx = jnp.min(jnp.where(v >= m, lanes, C), axis=1,
                      keepdims=True)
        selv = jnp.where(lanes_k == j, idx, selv)
        v = jnp.where(lanes == idx, -jnp.inf, v)
        return v, selv

    _, selv = lax.fori_loop(0, K, body,
                            (v, jnp.zeros((B, K), jnp.int32)))
    sel_ref[...] = selv


def _sc_gather_body(corr_hbm, ma_hbm, sel_hbm, out_hbm, sel_v, x_v, ma_v,
                    o_v):
    w = lax.axis_index("s") * 2 + lax.axis_index("c")
    pltpu.sync_copy(sel_hbm, sel_v)
    iota = lax.broadcasted_iota(jnp.int32, (16,), 0)
    ma_s = iota // 2
    ma_c = iota % 2

    def tile(t, _):
        r0 = t * 8
        b = t // (NTIL // B)
        pltpu.sync_copy(corr_hbm.at[pl.ds(r0, 8)], x_v)
        pltpu.sync_copy(ma_hbm.at[pl.ds(r0, 8)], ma_v)
        mvals = plsc.load_gather(ma_v, [ma_s, ma_c])
        plsc.store_scatter(o_v, [ma_s, ma_c], mvals)
        for k in range(K // 16):
            cidx = sel_v[pl.ds(b * K + k * 16, 16)]
            oidx = 2 + k * 16 + iota
            for s in range(8):
                sidx = jnp.full((16,), s, jnp.int32)
                vals = plsc.load_gather(x_v, [sidx, cidx])
                plsc.store_scatter(o_v, [sidx, oidx], vals)
        pltpu.sync_copy(o_v, out_hbm.at[pl.ds(r0, 8)])
        return _

    lax.fori_loop(w * TPW, (w + 1) * TPW, tile, 0)


@functools.cache
def _sc_gather():
    return pl.kernel(
        _sc_gather_body,
        out_type=jax.ShapeDtypeStruct((NPOS, 2 + K), jnp.float32),
        mesh=plsc.VectorSubcoreMesh(core_axis_name="c", subcore_axis_name="s"),
        scratch_types=[
            pltpu.VMEM((B * K,), jnp.int32),
            pltpu.VMEM((8, C), jnp.float32),
            pltpu.VMEM((8, 2), jnp.float32),
            pltpu.VMEM((8, 2 + K), jnp.float32),
        ],
        compiler_params=pltpu.CompilerParams(needs_layout_passes=False),
    )


@jax.jit
def kernel(corr, select_indices):
    corr_t = jnp.transpose(corr, (0, 2, 3, 1))
    ma, val = pl.pallas_call(
        _reduce_body,
        grid=(B, NH),
        in_specs=[pl.BlockSpec((1, HB, W, C), lambda b, i: (b, i, 0, 0))],
        out_specs=[
            pl.BlockSpec((1, HB, W, 2), lambda b, i: (b, i, 0, 0)),
            pl.BlockSpec((1, 1, C), lambda b, i: (b, 0, 0)),
        ],
        out_shape=[
            jax.ShapeDtypeStruct((B, H, W, 2), jnp.float32),
            jax.ShapeDtypeStruct((B, 1, C), jnp.float32),
        ],
    )(corr_t)

    sel = pl.pallas_call(
        _topk_body,
        grid=(1,),
        in_specs=[pl.BlockSpec((B, 1, C), lambda i: (0, 0, 0))],
        out_specs=pl.BlockSpec((B, K), lambda i: (0, 0)),
        out_shape=jax.ShapeDtypeStruct((B, K), jnp.int32),
    )(val)

    sel = jnp.take(sel, select_indices, axis=1)

    out2 = _sc_gather()(corr_t.reshape(NPOS, C), ma.reshape(NPOS, 2),
                        sel.reshape(B * K).astype(jnp.int32))
    out_t = out2.reshape(B, H, W, 2 + K)
    return jnp.transpose(out_t, (0, 3, 1, 2))


# speedup vs baseline: 6.9843x; 1.3263x over previous
import functools

import jax
import jax.numpy as jnp
from jax import lax
from jax.experimental import pallas as pl
from jax.experimental.pallas import tpu as pltpu
from jax.experimental.pallas import tpu_sc as plsc

B, C, H, W = 4, 3136, 56, 56
HB = 8
NH = H // HB
K = 64
NPOS = B * H * W
NTIL = NPOS // 8
NWORK = 32
TPW = NTIL // NWORK


def _reduce_body(x_ref, ma_ref, val_ref):
    i = pl.program_id(1)
    x = x_ref[0]
    mx = jnp.max(x, axis=2)
    sm = jnp.sum(x, axis=2) * (1.0 / C)
    ma_ref[0] = jnp.stack([mx, sm], axis=-1)
    pv = jnp.sum(x, axis=(0, 1))[None, :]

    @pl.when(i == 0)
    def _():
        val_ref[0] = pv

    @pl.when(i > 0)
    def _():
        val_ref[0] = val_ref[0] + pv


def _topk_body(val_ref, sel_ref):
    v = val_ref[:, 0, :]
    lanes = lax.broadcasted_iota(jnp.int32, (B, C), 1)
    lanes_k = lax.broadcasted_iota(jnp.int32, (B, K), 1)

    def body(j, carry):
        v, selv = carry
        m = jnp.max(v, axis=1, keepdims=True)
        idx = jnp.min(jnp.where(v >= m, lanes, C), axis=1,
                      keepdims=True)
        selv = jnp.where(lanes_k == j, idx, selv)
        v = jnp.where(lanes == idx, -jnp.inf, v)
        return v, selv

    _, selv = lax.fori_loop(0, K, body,
                            (v, jnp.zeros((B, K), jnp.int32)))
    sel_ref[...] = selv


def _sc_gather_body(corr_hbm, ma_hbm, sel_hbm, out_hbm, sel_v, xa_v, xb_v,
                    ma_v, o_v, sa, sb):
    w = lax.axis_index("s") * 2 + lax.axis_index("c")
    b = w // 8
    base = w * (NPOS // NWORK)
    pltpu.sync_copy(sel_hbm, sel_v)
    iota = lax.broadcasted_iota(jnp.int32, (16,), 0)
    ma_s = iota >> 1
    ma_c = iota & 1
    cidx = [sel_v[pl.ds(b * K + k * 16, 16)] for k in range(K // 16)]

    def start(m, buf, sem):
        pltpu.make_async_copy(
            corr_hbm.at[pl.ds(base + m * 8, 8)], buf, sem).start()

    def finish(m, buf, sem):
        pltpu.make_async_copy(
            corr_hbm.at[pl.ds(base + m * 8, 8)], buf, sem).wait()
        g = m % 7
        for k in range(K // 16):
            oidx = 2 + k * 16 + iota
            for s in range(8):
                sidx = jnp.full((16,), g * 8 + s, jnp.int32)
                vals = plsc.load_gather(buf, [jnp.full((16,), s, jnp.int32),
                                              cidx[k]])
                plsc.store_scatter(o_v, [sidx, oidx], vals)

    def step(m, carry):
        @pl.when(m % 7 == 0)
        def _():
            pltpu.sync_copy(ma_hbm.at[pl.ds(base + (m // 7) * 56, 56)], ma_v)
            for q in range(7):
                mvals = plsc.load_gather(ma_v, [q * 8 + ma_s, ma_c])
                plsc.store_scatter(o_v, [q * 8 + ma_s, ma_c], mvals)

        @pl.when(m < TPW - 1)
        def _():
            @pl.when(m % 2 == 0)
            def _():
                start(m + 1, xb_v, sb)

            @pl.when(m % 2 == 1)
            def _():
                start(m + 1, xa_v, sa)

        @pl.when(m % 2 == 0)
        def _():
            finish(m, xa_v, sa)

        @pl.when(m % 2 == 1)
        def _():
            finish(m, xb_v, sb)

        @pl.when(m % 7 == 6)
        def _():
            pltpu.sync_copy(o_v, out_hbm.at[pl.ds(base + (m // 7) * 56, 56)])
        return carry

    start(0, xa_v, sa)
    lax.fori_loop(0, TPW, step, 0)


@functools.cache
def _sc_gather():
    return pl.kernel(
        _sc_gather_body,
        out_type=jax.ShapeDtypeStruct((NPOS, 2 + K), jnp.float32),
        mesh=plsc.VectorSubcoreMesh(core_axis_name="c", subcore_axis_name="s"),
        scratch_types=[
            pltpu.VMEM((B * K,), jnp.int32),
            pltpu.VMEM((8, C), jnp.float32),
            pltpu.VMEM((8, C), jnp.float32),
            pltpu.VMEM((56, 2), jnp.float32),
            pltpu.VMEM((56, 2 + K), jnp.float32),
            pltpu.SemaphoreType.DMA,
            pltpu.SemaphoreType.DMA,
        ],
        compiler_params=pltpu.CompilerParams(needs_layout_passes=False),
    )


@jax.jit
def kernel(corr, select_indices):
    corr_t = jnp.transpose(corr, (0, 2, 3, 1))
    ma, val = pl.pallas_call(
        _reduce_body,
        grid=(B, NH),
        in_specs=[pl.BlockSpec((1, HB, W, C), lambda b, i: (b, i, 0, 0))],
        out_specs=[
            pl.BlockSpec((1, HB, W, 2), lambda b, i: (b, i, 0, 0)),
            pl.BlockSpec((1, 1, C), lambda b, i: (b, 0, 0)),
        ],
        out_shape=[
            jax.ShapeDtypeStruct((B, H, W, 2), jnp.float32),
            jax.ShapeDtypeStruct((B, 1, C), jnp.float32),
        ],
    )(corr_t)

    sel = pl.pallas_call(
        _topk_body,
        grid=(1,),
        in_specs=[pl.BlockSpec((B, 1, C), lambda i: (0, 0, 0))],
        out_specs=pl.BlockSpec((B, K), lambda i: (0, 0)),
        out_shape=jax.ShapeDtypeStruct((B, K), jnp.int32),
    )(val)

    sel = jnp.take(sel, select_indices, axis=1)

    out2 = _sc_gather()(corr_t.reshape(NPOS, C), ma.reshape(NPOS, 2),
                        sel.reshape(B * K).astype(jnp.int32))
    out_t = out2.reshape(B, H, W, 2 + K)
    return jnp.transpose(out_t, (0, 3, 1, 2))
